# baseline (speedup 1.0000x reference)
import jax
import jax.numpy as jnp
from jax.experimental import pallas as pl
from jax.experimental.pallas import tpu as pltpu

_TAU = 0.001
_G = 64
_GS = 16
_NG = 16
_NPN = 8
_GO = _NG * _NPN


def _fwd_kernel(x_ref, w_ref, o_ref, p_ref):
    @pl.when(pl.program_id(0) == 0)
    def _():
        logits = w_ref[...] * (1.0 / _TAU)
        m = jnp.max(logits, axis=1, keepdims=True)
        e = jnp.exp(logits - m)
        p = e / jnp.sum(e, axis=1, keepdims=True)
        p_ref[...] = p.astype(jnp.bfloat16)

    x = x_ref[...].astype(jnp.bfloat16)
    for g in range(_G):
        o_ref[:, g * _GO:(g + 1) * _GO] = jnp.dot(
            x[:, g * _GS:(g + 1) * _GS], p_ref[g],
            preferred_element_type=jnp.float32)


def kernel(x, W):
    B = x.shape[0]
    BB = 256
    wt = jnp.transpose(W, (0, 3, 1, 2)).reshape(_G, _GS, _GO)
    out = pl.pallas_call(
        _fwd_kernel,
        grid=(B // BB,),
        in_specs=[
            pl.BlockSpec((BB, _G * _GS), lambda i: (i, 0)),
            pl.BlockSpec((_G, _GS, _GO), lambda i: (0, 0, 0)),
        ],
        out_specs=pl.BlockSpec((BB, _G * _GO), lambda i: (i, 0)),
        out_shape=jax.ShapeDtypeStruct((B, _G * _GO), jnp.float32),
        scratch_shapes=[pltpu.VMEM((_G, _GS, _GO), jnp.bfloat16)],
    )(x, wt)
    return out.reshape(B, _G * _NG, _NPN)


# speedup vs baseline: 1.9343x; 1.9343x over previous
import jax
import jax.numpy as jnp
from jax.experimental import pallas as pl
from jax.experimental.pallas import tpu as pltpu

_TAU = 0.001
_G = 64
_GS = 16
_NG = 16
_NPN = 8
_NC = 8


def _fwd_kernel(x_ref, w_ref, o_ref, m_ref):
    @pl.when(pl.program_id(0) == 0)
    def _():
        logits = w_ref[...] * (1.0 / _TAU)
        mx = jnp.max(logits, axis=2, keepdims=True)
        e = jnp.exp(logits - mx)
        p = (e / jnp.sum(e, axis=2, keepdims=True)).astype(jnp.bfloat16)
        rows = jax.lax.broadcasted_iota(jnp.int32, (128, 128), 0)
        cols = jax.lax.broadcasted_iota(jnp.int32, (128, 128), 1)
        mask = (rows // _GS) == (cols // _GS)
        zero = jnp.zeros((128, 128), jnp.bfloat16)
        for c in range(_NC):
            for n in range(_NPN):
                s = p[n, 8 * c:8 * c + 8].reshape(128, _GS)
                t = jnp.tile(s, (1, 8))
                bd = jnp.where(mask, t, zero)
                m_ref[c, n // 2, :, (n % 2) * 128:(n % 2) * 128 + 128] = bd

    x = x_ref[...].astype(jnp.bfloat16)
    bb = x.shape[0]
    for c in range(_NC):
        xc = x[:, c * 128:(c + 1) * 128]
        parts = []
        for k in range(_NPN // 2):
            r = jnp.dot(xc, m_ref[c, k], preferred_element_type=jnp.float32)
            parts.append(r[:, :128])
            parts.append(r[:, 128:])
        v = jnp.stack(parts, axis=1).reshape(bb * _NPN, 128)
        o_ref[:, c * 128:(c + 1) * 128] = v


def kernel(x, W):
    B = x.shape[0]
    BB = 256
    wn = jnp.transpose(W, (2, 0, 3, 1))
    out2d = pl.pallas_call(
        _fwd_kernel,
        grid=(B // BB,),
        in_specs=[
            pl.BlockSpec((BB, _G * _GS), lambda i: (i, 0)),
            pl.BlockSpec((_NPN, _G, _GS, _NG), lambda i: (0, 0, 0, 0)),
        ],
        out_specs=pl.BlockSpec((BB * _NPN, 1024), lambda i: (i, 0)),
        out_shape=jax.ShapeDtypeStruct((B * _NPN, 1024), jnp.float32),
        scratch_shapes=[pltpu.VMEM((_NC, _NPN // 2, 128, 256), jnp.bfloat16)],
    )(x, wn)
    return jnp.transpose(out2d.reshape(B, _NPN, 1024), (0, 2, 1))


# speedup vs baseline: 4.8014x; 2.4823x over previous
import jax
import jax.numpy as jnp
from jax.experimental import pallas as pl
from jax.experimental.pallas import tpu as pltpu

_TAU = 0.001
_G = 64
_GS = 16
_NG = 16
_NPN = 8
_NC = 8
_BB = 256


def _copy(buf_ref, o_ref, sem, slot, n, step):
    return pltpu.make_async_copy(
        buf_ref.at[slot, n],
        o_ref.at[pl.ds(step * _BB, _BB), n, :],
        sem.at[slot, n])


def _fwd_kernel(x_ref, w_ref, o_ref, buf_ref, m_ref, sem):
    i = pl.program_id(0)
    nsteps = pl.num_programs(0)
    slot = jax.lax.rem(i, 2)

    @pl.when(i == 0)
    def _():
        logits = w_ref[...] * (1.0 / _TAU)
        mx = jnp.max(logits, axis=2, keepdims=True)
        e = jnp.exp(logits - mx)
        p = (e / jnp.sum(e, axis=2, keepdims=True)).astype(jnp.bfloat16)
        rows = jax.lax.broadcasted_iota(jnp.int32, (128, 128), 0)
        cols = jax.lax.broadcasted_iota(jnp.int32, (128, 128), 1)
        mask = (rows // _GS) == (cols // _GS)
        zero = jnp.zeros((128, 128), jnp.bfloat16)
        for c in range(_NC):
            for n in range(_NPN):
                s = p[n, 8 * c:8 * c + 8].reshape(128, _GS)
                t = jnp.tile(s, (1, 8))
                bd = jnp.where(mask, t, zero)
                m_ref[c, n // 2, :, (n % 2) * 128:(n % 2) * 128 + 128] = bd

    @pl.when(i >= 2)
    def _():
        for n in range(_NPN):
            _copy(buf_ref, o_ref, sem, slot, n, i - 2).wait()

    x = x_ref[...].astype(jnp.bfloat16)
    for c in range(_NC):
        xc = x[:, c * 128:(c + 1) * 128]
        for k in range(_NPN // 2):
            r = jnp.dot(xc, m_ref[c, k], preferred_element_type=jnp.float32)
            buf_ref[slot, 2 * k, :, c * 128:(c + 1) * 128] = r[:, :128]
            buf_ref[slot, 2 * k + 1, :, c * 128:(c + 1) * 128] = r[:, 128:]

    for n in range(_NPN):
        _copy(buf_ref, o_ref, sem, slot, n, i).start()

    @pl.when(i == nsteps - 1)
    def _():
        for n in range(_NPN):
            _copy(buf_ref, o_ref, sem, 1 - slot, n, i - 1).wait()
        for n in range(_NPN):
            _copy(buf_ref, o_ref, sem, slot, n, i).wait()


def kernel(x, W):
    B = x.shape[0]
    wn = jnp.transpose(W, (2, 0, 3, 1))
    out3 = pl.pallas_call(
        _fwd_kernel,
        grid=(B // _BB,),
        in_specs=[
            pl.BlockSpec((_BB, _G * _GS), lambda i: (i, 0)),
            pl.BlockSpec((_NPN, _G, _GS, _NG), lambda i: (0, 0, 0, 0)),
        ],
        out_specs=pl.BlockSpec(memory_space=pltpu.MemorySpace.HBM),
        out_shape=jax.ShapeDtypeStruct((B, _NPN, 1024), jnp.float32),
        scratch_shapes=[
            pltpu.VMEM((2, _NPN, _BB, 1024), jnp.float32),
            pltpu.VMEM((_NC, _NPN // 2, 128, 256), jnp.bfloat16),
            pltpu.SemaphoreType.DMA((2, _NPN)),
        ],
    )(x, wn)
    return jnp.transpose(out3, (0, 2, 1))
